# Initial kernel scaffold; baseline (speedup 1.0000x reference)
import jax
import jax.numpy as jnp
from jax.experimental import pallas as pl


def kernel(points, cp, voxel, plane, quat):
    raise NotImplementedError("write your pallas kernel here")


# speedup vs baseline: 4.4318x; 4.4318x over previous
import functools

import jax
import jax.numpy as jnp
from jax import lax
from jax.experimental import pallas as pl
from jax.experimental.pallas import tpu as pltpu
from jax.experimental.pallas import tpu_sc as plsc

_GRID_BOUND = 0.5
_GRID_SIZE = 32
_NC = 2
_NS = 16
_NW = _NC * _NS
_L = 16
_UNROLL = 4


def _sc_loss_kernel(n_points: int):
    g3 = _GRID_SIZE ** 3
    grid_min = -_GRID_BOUND + _GRID_BOUND / _GRID_SIZE
    scale = _GRID_SIZE / (2.0 * _GRID_BOUND)
    n_groups = n_points // _L
    mesh = plsc.VectorSubcoreMesh(
        core_axis_name="c", subcore_axis_name="s",
        num_cores=_NC, num_subcores=_NS)

    @functools.partial(
        pl.kernel,
        out_type=jax.ShapeDtypeStruct((_NW * 2 * _L,), jnp.float32),
        mesh=mesh,
        scratch_types=[
            pltpu.VMEM((3 * g3,), jnp.float32),
            pltpu.VMEM((3 * n_points,), jnp.float32),
            pltpu.VMEM((7 * 12 * _L,), jnp.float32),
            pltpu.VMEM((2 * _L,), jnp.float32),
            pltpu.SemaphoreType.DMA,
        ],
        compiler_params=pltpu.CompilerParams(needs_layout_passes=False),
    )
    def body(cp_hbm, pts_hbm, par_hbm, out_hbm,
             cp_v, pts_v, par_v, acc_v, sem):
        cid = lax.axis_index("c")
        sid = lax.axis_index("s")
        wid = sid * _NC + cid
        batch = wid

        copies = [
            pltpu.async_copy(cp_hbm.at[pl.ds(batch * 3 * g3, 3 * g3)],
                             cp_v, sem),
            pltpu.async_copy(
                pts_hbm.at[pl.ds(batch * 3 * n_points, 3 * n_points)],
                pts_v, sem),
            pltpu.async_copy(par_hbm, par_v, sem),
        ]
        for c in copies:
            c.wait()

        zero = jnp.zeros((_L,), jnp.float32)

        for t in range(7):
            a = [par_v[pl.ds((t * 12 + j) * _L, _L)] for j in range(12)]

            def group_body(i, part, a=a):
                for u in range(_UNROLL):
                    o = (i * _UNROLL + u) * _L
                    px = pts_v[pl.ds(o, _L)]
                    py = pts_v[pl.ds(n_points + o, _L)]
                    pz = pts_v[pl.ds(2 * n_points + o, _L)]
                    tx = a[0] * px + a[1] * py + a[2] * pz + a[9]
                    ty = a[3] * px + a[4] * py + a[5] * pz + a[10]
                    tz = a[6] * px + a[7] * py + a[8] * pz + a[11]
                    ix = jnp.clip((tx - grid_min) * scale, 0.0, 31.0)
                    iy = jnp.clip((ty - grid_min) * scale, 0.0, 31.0)
                    iz = jnp.clip((tz - grid_min) * scale, 0.0, 31.0)
                    ixi = (ix + 0.5).astype(jnp.int32)
                    iyi = (iy + 0.5).astype(jnp.int32)
                    izi = (iz + 0.5).astype(jnp.int32)
                    cell = ixi * 1024 + iyi * 32 + izi
                    cx = plsc.load_gather(cp_v, [cell])
                    cy = plsc.load_gather(cp_v, [cell + g3])
                    cz = plsc.load_gather(cp_v, [cell + 2 * g3])
                    m = 1.0 - (plsc.bitcast(cx, jnp.int32) & 1).astype(
                        jnp.float32)
                    dx = tx - cx
                    dy = ty - cy
                    dz = tz - cz
                    part = part + (dx * dx + dy * dy + dz * dz) * m
                return part

            part = lax.fori_loop(0, n_groups // _UNROLL, group_body, zero)
            if t == 0:
                acc_v[pl.ds(0, _L)] = part
            elif t == 3:
                acc_v[pl.ds(_L, _L)] = part
            else:
                plsc.addupdate(acc_v.at[pl.ds(0 if t < 3 else _L, _L)], part)

        pltpu.sync_copy(acc_v, out_hbm.at[pl.ds(wid * 2 * _L, 2 * _L)])

    return body


def kernel(points, cp, voxel, plane, quat):
    B, N, _ = points.shape
    g3 = _GRID_SIZE ** 3

    pn = plane[:, :3] / (jnp.linalg.norm(plane[:, :3], axis=1, keepdims=True)
                         + 1e-12)
    a_ref = jnp.eye(3, dtype=jnp.float32)[None] - 2.0 * pn[:, :, None] * pn[:, None, :]
    t_ref = -2.0 * plane[:, 3:4] * pn
    q = quat / (jnp.linalg.norm(quat, axis=1, keepdims=True) + 1e-12)
    w, x, y, z = q[:, 0], q[:, 1], q[:, 2], q[:, 3]
    rot = jnp.stack([
        jnp.stack([1 - 2 * (y * y + z * z), 2 * (x * y - w * z), 2 * (x * z + w * y)], axis=-1),
        jnp.stack([2 * (x * y + w * z), 1 - 2 * (x * x + z * z), 2 * (y * z - w * x)], axis=-1),
        jnp.stack([2 * (x * z - w * y), 2 * (y * z + w * x), 1 - 2 * (x * x + y * y)], axis=-1),
    ], axis=1)
    amat = jnp.concatenate([a_ref, rot], axis=0).reshape(7, 9)
    tvec = jnp.concatenate([t_ref, jnp.zeros((4, 3), jnp.float32)], axis=0)
    params = jnp.concatenate([amat, tvec], axis=1)
    params = jnp.broadcast_to(params[:, :, None], (7, 12, _L))
    params = params.astype(jnp.float32).reshape(7 * 12 * _L)

    vox_bit = voxel.reshape(B, g3).astype(jnp.int32)
    cpx_enc = jax.lax.bitcast_convert_type(
        (jax.lax.bitcast_convert_type(cp[:, :, 0], jnp.int32) & ~1) | vox_bit,
        jnp.float32)
    cp_t = jnp.stack([cpx_enc, cp[:, :, 1], cp[:, :, 2]], axis=1)
    cp_t = cp_t.reshape(B * 3 * g3)

    pts_t = jnp.swapaxes(points, 1, 2).reshape(B * 3 * N)

    out = _sc_loss_kernel(N)(cp_t, pts_t, params)
    out = out.reshape(_NW, 2, _L)
    ref_loss = (jnp.sum(out[:, 0, :]) / B).reshape(1)
    rot_loss = (jnp.sum(out[:, 1, :]) / B).reshape(1)
    return (ref_loss, rot_loss)
